# baseline (speedup 1.0000x reference)
import functools

import jax
import jax.numpy as jnp
from jax import lax
from jax.experimental import pallas as pl
from jax.experimental.pallas import tpu as pltpu
from jax.experimental.pallas import tpu_sc as plsc

_NC = 2
_NS = 16


def _sc_gather(table_pad, idx):
    n = idx.shape[0]
    d = table_pad.shape[1]
    nw = _NC * _NS
    b_per_w = n // nw
    assert n % (8 * nw) == 0

    mesh = plsc.VectorSubcoreMesh(
        core_axis_name="c", subcore_axis_name="s",
        num_cores=_NC, num_subcores=_NS)

    @functools.partial(
        pl.kernel, mesh=mesh,
        out_type=jax.ShapeDtypeStruct((n, d), jnp.float32),
        scratch_types=[
            pltpu.VMEM((b_per_w,), jnp.int32),
            pltpu.VMEM((b_per_w, d), jnp.float32),
            pltpu.SemaphoreType.DMA,
        ],
    )
    def gather_kernel(table_hbm, idx_hbm, out_hbm, idx_v, rows_v, sem):
        wid = lax.axis_index("s") * _NC + lax.axis_index("c")
        base = wid * b_per_w
        pltpu.sync_copy(idx_hbm.at[pl.ds(base, b_per_w)], idx_v)
        pltpu.async_copy(table_hbm.at[idx_v], rows_v, sem).wait()
        pltpu.sync_copy(rows_v, out_hbm.at[pl.ds(base, b_per_w)])

    return gather_kernel(table_pad, idx)


def _eye(n, dtype=jnp.float32):
    r = lax.broadcasted_iota(jnp.int32, (n, n), 0)
    c = lax.broadcasted_iota(jnp.int32, (n, n), 1)
    return (r == c).astype(dtype)


def _tpad_body(emb, tt_ref, out_ref):
    t = lax.dot_general(tt_ref[...], _eye(emb), (((0,), (0,)), ((), ())),
                        preferred_element_type=jnp.float32)
    out_ref[...] = jnp.concatenate(
        [t, jnp.zeros(t.shape, jnp.float32)], axis=1)


def _transpose_pad(tableT, block_t=4096):
    emb, v = tableT.shape
    nt = pl.cdiv(v, block_t)
    return pl.pallas_call(
        functools.partial(_tpad_body, emb),
        grid=(nt,),
        in_specs=[pl.BlockSpec((emb, block_t), lambda i: (0, i))],
        out_specs=pl.BlockSpec((block_t, 2 * emb), lambda i: (i, 0)),
        out_shape=jax.ShapeDtypeStruct((v, 2 * emb), jnp.float32),
    )(tableT)


def _mlp_body(ctx, emb, b, bv, g_ref, w1_ref, b1_ref, w2t_ref, b2_ref,
              out_ref, hid_ref, eye_ref):
    @pl.when(pl.program_id(0) == 0)
    def _():
        h = b1_ref[...].astype(jnp.float32)
        for c in range(ctx):
            h = h + jnp.dot(g_ref[c * b:(c + 1) * b, 0:emb],
                            w1_ref[c * emb:(c + 1) * emb, :],
                            preferred_element_type=jnp.float32)
        hid_ref[...] = jnp.maximum(h, 0.0).astype(jnp.bfloat16)
        eye_ref[...] = _eye(bv)

    acc = lax.dot_general(w2t_ref[...].astype(jnp.bfloat16), hid_ref[...],
                          (((1,), (1,)), ((), ())),
                          preferred_element_type=jnp.float32)
    b2col = lax.dot_general(eye_ref[...], b2_ref[...],
                            (((1,), (1,)), ((), ())),
                            preferred_element_type=jnp.float32)
    out_ref[...] = acc + b2col


def _mlp(g, W1, b1_row, W2T, b2_row, ctx, emb, block_v=512):
    n, dpad = g.shape
    b = n // ctx
    k1, h = W1.shape
    v = W2T.shape[0]
    nv = pl.cdiv(v, block_v)
    return pl.pallas_call(
        functools.partial(_mlp_body, ctx, emb, b, block_v),
        grid=(nv,),
        in_specs=[
            pl.BlockSpec((n, dpad), lambda i: (0, 0)),
            pl.BlockSpec((k1, h), lambda i: (0, 0)),
            pl.BlockSpec((1, h), lambda i: (0, 0)),
            pl.BlockSpec((block_v, h), lambda i: (i, 0)),
            pl.BlockSpec((1, block_v), lambda i: (0, i)),
        ],
        out_specs=pl.BlockSpec((block_v, b), lambda i: (i, 0)),
        out_shape=jax.ShapeDtypeStruct((v, b), jnp.float32),
        scratch_shapes=[pltpu.VMEM((b, h), jnp.bfloat16),
                        pltpu.VMEM((block_v, block_v), jnp.float32)],
    )(g, W1, b1_row, W2T, b2_row)


def kernel(x, table, W1, b1, W2, b2):
    b, ctx = x.shape
    emb = table.shape[1]
    idx = x.T.reshape(-1).astype(jnp.int32)
    table_pad = _transpose_pad(table.T)
    g = _sc_gather(table_pad, idx)
    out_t = _mlp(g, W1, b1.reshape(1, -1), W2.T, b2.reshape(1, -1), ctx, emb)
    return out_t.T


# speedup vs baseline: 1.0262x; 1.0262x over previous
import functools

import jax
import jax.numpy as jnp
from jax import lax
from jax.experimental import pallas as pl
from jax.experimental.pallas import tpu as pltpu
from jax.experimental.pallas import tpu_sc as plsc

_NC = 2
_NS = 16


def _sc_gather(table_pad, idx):
    n = idx.shape[0]
    d = table_pad.shape[1]
    nw = _NC * _NS
    b_per_w = n // nw
    assert n % (8 * nw) == 0

    mesh = plsc.VectorSubcoreMesh(
        core_axis_name="c", subcore_axis_name="s",
        num_cores=_NC, num_subcores=_NS)

    @functools.partial(
        pl.kernel, mesh=mesh,
        out_type=jax.ShapeDtypeStruct((n, d), jnp.float32),
        scratch_types=[
            pltpu.VMEM((b_per_w,), jnp.int32),
            pltpu.VMEM((b_per_w, d), jnp.float32),
            pltpu.SemaphoreType.DMA,
        ],
    )
    def gather_kernel(table_hbm, idx_hbm, out_hbm, idx_v, rows_v, sem):
        wid = lax.axis_index("s") * _NC + lax.axis_index("c")
        base = wid * b_per_w
        pltpu.sync_copy(idx_hbm.at[pl.ds(base, b_per_w)], idx_v)
        pltpu.async_copy(table_hbm.at[idx_v], rows_v, sem).wait()
        pltpu.sync_copy(rows_v, out_hbm.at[pl.ds(base, b_per_w)])

    return gather_kernel(table_pad, idx)


def _eye(n, dtype=jnp.float32):
    r = lax.broadcasted_iota(jnp.int32, (n, n), 0)
    c = lax.broadcasted_iota(jnp.int32, (n, n), 1)
    return (r == c).astype(dtype)


def _tpad_body(emb, tt_ref, out_ref):
    t = lax.dot_general(tt_ref[...], _eye(emb), (((0,), (0,)), ((), ())),
                        preferred_element_type=jnp.float32)
    out_ref[...] = jnp.concatenate(
        [t, jnp.zeros(t.shape, jnp.float32)], axis=1)


def _transpose_pad(tableT, block_t=8192):
    emb, v = tableT.shape
    nt = pl.cdiv(v, block_t)
    return pl.pallas_call(
        functools.partial(_tpad_body, emb),
        grid=(nt,),
        in_specs=[pl.BlockSpec((emb, block_t), lambda i: (0, i))],
        out_specs=pl.BlockSpec((block_t, 2 * emb), lambda i: (i, 0)),
        out_shape=jax.ShapeDtypeStruct((v, 2 * emb), jnp.float32),
    )(tableT)


def _mlp_body(ctx, emb, b, bv, g_ref, w1_ref, b1_ref, w2t_ref, b2_ref,
              out_ref, hid_ref, eye_ref):
    @pl.when(pl.program_id(0) == 0)
    def _():
        h = b1_ref[...].astype(jnp.float32)
        for c in range(ctx):
            h = h + jnp.dot(g_ref[c * b:(c + 1) * b, 0:emb],
                            w1_ref[c * emb:(c + 1) * emb, :],
                            preferred_element_type=jnp.float32)
        hid_ref[...] = jnp.maximum(h, 0.0)
        eye_ref[...] = _eye(bv)

    acc = lax.dot_general(w2t_ref[...], hid_ref[...],
                          (((1,), (1,)), ((), ())),
                          preferred_element_type=jnp.float32)
    b2col = lax.dot_general(eye_ref[...], b2_ref[...],
                            (((1,), (1,)), ((), ())),
                            preferred_element_type=jnp.float32)
    out_ref[...] = acc + b2col


def _mlp(g, W1, b1_row, W2T, b2_row, ctx, emb, block_v=1024):
    n, dpad = g.shape
    b = n // ctx
    k1, h = W1.shape
    v = W2T.shape[0]
    nv = pl.cdiv(v, block_v)
    return pl.pallas_call(
        functools.partial(_mlp_body, ctx, emb, b, block_v),
        grid=(nv,),
        in_specs=[
            pl.BlockSpec((n, dpad), lambda i: (0, 0)),
            pl.BlockSpec((k1, h), lambda i: (0, 0)),
            pl.BlockSpec((1, h), lambda i: (0, 0)),
            pl.BlockSpec((block_v, h), lambda i: (i, 0)),
            pl.BlockSpec((1, block_v), lambda i: (0, i)),
        ],
        out_specs=pl.BlockSpec((block_v, b), lambda i: (i, 0)),
        out_shape=jax.ShapeDtypeStruct((v, b), jnp.float32),
        scratch_shapes=[pltpu.VMEM((b, h), jnp.float32),
                        pltpu.VMEM((block_v, block_v), jnp.float32)],
    )(g, W1, b1_row, W2T, b2_row)


def kernel(x, table, W1, b1, W2, b2):
    b, ctx = x.shape
    emb = table.shape[1]
    idx = x.T.reshape(-1).astype(jnp.int32)
    table_pad = _transpose_pad(table.T)
    g = _sc_gather(table_pad, idx)
    out_t = _mlp(g, W1, b1.reshape(1, -1), W2.T, b2.reshape(1, -1), ctx, emb)
    return out_t.T


# speedup vs baseline: 1.0302x; 1.0039x over previous
import functools

import jax
import jax.numpy as jnp
from jax import lax
from jax.experimental import pallas as pl
from jax.experimental.pallas import tpu as pltpu
from jax.experimental.pallas import tpu_sc as plsc

_NC = 2
_NS = 16


def _sc_gather(table_pad, idx):
    n = idx.shape[0]
    d = table_pad.shape[1]
    nw = _NC * _NS
    b_per_w = n // nw
    assert n % (8 * nw) == 0

    mesh = plsc.VectorSubcoreMesh(
        core_axis_name="c", subcore_axis_name="s",
        num_cores=_NC, num_subcores=_NS)

    @functools.partial(
        pl.kernel, mesh=mesh,
        out_type=jax.ShapeDtypeStruct((n, d), jnp.float32),
        scratch_types=[
            pltpu.VMEM((b_per_w,), jnp.int32),
            pltpu.VMEM((b_per_w, d), jnp.float32),
            pltpu.SemaphoreType.DMA,
        ],
    )
    def gather_kernel(table_hbm, idx_hbm, out_hbm, idx_v, rows_v, sem):
        wid = lax.axis_index("s") * _NC + lax.axis_index("c")
        base = wid * b_per_w
        pltpu.sync_copy(idx_hbm.at[pl.ds(base, b_per_w)], idx_v)
        pltpu.async_copy(table_hbm.at[idx_v], rows_v, sem).wait()
        pltpu.sync_copy(rows_v, out_hbm.at[pl.ds(base, b_per_w)])

    return gather_kernel(table_pad, idx)


def _eye(n, dtype=jnp.float32):
    r = lax.broadcasted_iota(jnp.int32, (n, n), 0)
    c = lax.broadcasted_iota(jnp.int32, (n, n), 1)
    return (r == c).astype(dtype)


def _tpad_body(emb, tt_ref, out_ref):
    t = lax.dot_general(tt_ref[...], _eye(emb), (((0,), (0,)), ((), ())),
                        preferred_element_type=jnp.float32)
    out_ref[...] = jnp.concatenate(
        [t, jnp.zeros(t.shape, jnp.float32)], axis=1)


def _transpose_pad(tableT, block_t=16384):
    emb, v = tableT.shape
    nt = pl.cdiv(v, block_t)
    return pl.pallas_call(
        functools.partial(_tpad_body, emb),
        grid=(nt,),
        in_specs=[pl.BlockSpec((emb, block_t), lambda i: (0, i))],
        out_specs=pl.BlockSpec((block_t, 2 * emb), lambda i: (i, 0)),
        out_shape=jax.ShapeDtypeStruct((v, 2 * emb), jnp.float32),
    )(tableT)


def _mlp_body(ctx, emb, b, bv, g_ref, w1_ref, b1_ref, w2t_ref, b2_ref,
              out_ref, hid_ref, eye_ref):
    @pl.when(pl.program_id(0) == 0)
    def _():
        h = b1_ref[...].astype(jnp.float32)
        for c in range(ctx):
            h = h + jnp.dot(g_ref[c * b:(c + 1) * b, 0:emb],
                            w1_ref[c * emb:(c + 1) * emb, :],
                            preferred_element_type=jnp.float32)
        hid_ref[...] = jnp.maximum(h, 0.0)
        eye_ref[...] = _eye(bv)

    acc = lax.dot_general(w2t_ref[...], hid_ref[...],
                          (((1,), (1,)), ((), ())),
                          preferred_element_type=jnp.float32)
    b2col = lax.dot_general(eye_ref[...], b2_ref[...],
                            (((1,), (1,)), ((), ())),
                            preferred_element_type=jnp.float32)
    out_ref[...] = acc + b2col


def _mlp(g, W1, b1_row, W2T, b2_row, ctx, emb, block_v=1024):
    n, dpad = g.shape
    b = n // ctx
    k1, h = W1.shape
    v = W2T.shape[0]
    nv = pl.cdiv(v, block_v)
    return pl.pallas_call(
        functools.partial(_mlp_body, ctx, emb, b, block_v),
        grid=(nv,),
        in_specs=[
            pl.BlockSpec((n, dpad), lambda i: (0, 0)),
            pl.BlockSpec((k1, h), lambda i: (0, 0)),
            pl.BlockSpec((1, h), lambda i: (0, 0)),
            pl.BlockSpec((block_v, h), lambda i: (i, 0)),
            pl.BlockSpec((1, block_v), lambda i: (0, i)),
        ],
        out_specs=pl.BlockSpec((block_v, b), lambda i: (i, 0)),
        out_shape=jax.ShapeDtypeStruct((v, b), jnp.float32),
        scratch_shapes=[pltpu.VMEM((b, h), jnp.float32),
                        pltpu.VMEM((block_v, block_v), jnp.float32)],
    )(g, W1, b1_row, W2T, b2_row)


def kernel(x, table, W1, b1, W2, b2):
    b, ctx = x.shape
    emb = table.shape[1]
    idx = x.T.reshape(-1).astype(jnp.int32)
    table_pad = _transpose_pad(table.T)
    g = _sc_gather(table_pad, idx)
    out_t = _mlp(g, W1, b1.reshape(1, -1), W2.T, b2.reshape(1, -1), ctx, emb)
    return out_t.T


# speedup vs baseline: 1.0376x; 1.0071x over previous
import functools

import jax
import jax.numpy as jnp
from jax import lax
from jax.experimental import pallas as pl
from jax.experimental.pallas import tpu as pltpu
from jax.experimental.pallas import tpu_sc as plsc

_NC = 2
_NS = 16


def _sc_gather(table_pad, idx):
    n = idx.shape[0]
    d = table_pad.shape[1]
    nw = _NC * _NS
    b_per_w = n // nw
    assert n % (8 * nw) == 0

    mesh = plsc.VectorSubcoreMesh(
        core_axis_name="c", subcore_axis_name="s",
        num_cores=_NC, num_subcores=_NS)

    @functools.partial(
        pl.kernel, mesh=mesh,
        out_type=jax.ShapeDtypeStruct((n, d), jnp.float32),
        scratch_types=[
            pltpu.VMEM((b_per_w,), jnp.int32),
            pltpu.VMEM((b_per_w, d), jnp.float32),
            pltpu.SemaphoreType.DMA,
        ],
    )
    def gather_kernel(table_hbm, idx_hbm, out_hbm, idx_v, rows_v, sem):
        wid = lax.axis_index("s") * _NC + lax.axis_index("c")
        base = wid * b_per_w
        pltpu.sync_copy(idx_hbm.at[pl.ds(base, b_per_w)], idx_v)
        pltpu.async_copy(table_hbm.at[idx_v], rows_v, sem).wait()
        pltpu.sync_copy(rows_v, out_hbm.at[pl.ds(base, b_per_w)])

    return gather_kernel(table_pad, idx)


def _eye(n, dtype=jnp.float32):
    r = lax.broadcasted_iota(jnp.int32, (n, n), 0)
    c = lax.broadcasted_iota(jnp.int32, (n, n), 1)
    return (r == c).astype(dtype)


def _tpad_body(emb, tt_ref, out_ref):
    t = lax.dot_general(tt_ref[...], _eye(emb), (((0,), (0,)), ((), ())),
                        preferred_element_type=jnp.float32)
    out_ref[...] = jnp.concatenate(
        [t, jnp.zeros(t.shape, jnp.float32)], axis=1)


def _transpose_pad(tableT, block_t=25088):
    emb, v = tableT.shape
    nt = pl.cdiv(v, block_t)
    return pl.pallas_call(
        functools.partial(_tpad_body, emb),
        grid=(nt,),
        in_specs=[pl.BlockSpec((emb, block_t), lambda i: (0, i))],
        out_specs=pl.BlockSpec((block_t, 2 * emb), lambda i: (i, 0)),
        out_shape=jax.ShapeDtypeStruct((v, 2 * emb), jnp.float32),
    )(tableT)


def _mlp_body(ctx, emb, b, bv, g_ref, w1_ref, b1_ref, w2t_ref, b2_ref,
              out_ref, hid_ref, eye_ref):
    @pl.when(pl.program_id(0) == 0)
    def _():
        h = b1_ref[...].astype(jnp.float32)
        for c in range(ctx):
            h = h + jnp.dot(g_ref[c * b:(c + 1) * b, 0:emb],
                            w1_ref[c * emb:(c + 1) * emb, :],
                            preferred_element_type=jnp.float32)
        hid_ref[...] = jnp.maximum(h, 0.0)
        eye_ref[...] = _eye(bv)

    acc = lax.dot_general(w2t_ref[...], hid_ref[...],
                          (((1,), (1,)), ((), ())),
                          preferred_element_type=jnp.float32)
    b2col = lax.dot_general(eye_ref[...], b2_ref[...],
                            (((1,), (1,)), ((), ())),
                            preferred_element_type=jnp.float32)
    out_ref[...] = acc + b2col


def _mlp(g, W1, b1_row, W2T, b2_row, ctx, emb, block_v=1024):
    n, dpad = g.shape
    b = n // ctx
    k1, h = W1.shape
    v = W2T.shape[0]
    nv = pl.cdiv(v, block_v)
    return pl.pallas_call(
        functools.partial(_mlp_body, ctx, emb, b, block_v),
        grid=(nv,),
        in_specs=[
            pl.BlockSpec((n, dpad), lambda i: (0, 0)),
            pl.BlockSpec((k1, h), lambda i: (0, 0)),
            pl.BlockSpec((1, h), lambda i: (0, 0)),
            pl.BlockSpec((block_v, h), lambda i: (i, 0)),
            pl.BlockSpec((1, block_v), lambda i: (0, i)),
        ],
        out_specs=pl.BlockSpec((block_v, b), lambda i: (i, 0)),
        out_shape=jax.ShapeDtypeStruct((v, b), jnp.float32),
        scratch_shapes=[pltpu.VMEM((b, h), jnp.float32),
                        pltpu.VMEM((block_v, block_v), jnp.float32)],
    )(g, W1, b1_row, W2T, b2_row)


def kernel(x, table, W1, b1, W2, b2):
    b, ctx = x.shape
    emb = table.shape[1]
    idx = x.T.reshape(-1).astype(jnp.int32)
    table_pad = _transpose_pad(table.T)
    g = _sc_gather(table_pad, idx)
    out_t = _mlp(g, W1, b1.reshape(1, -1), W2.T, b2.reshape(1, -1), ctx, emb)
    return out_t.T
